# baseline (speedup 1.0000x reference)
import functools

import jax
import jax.numpy as jnp
from jax import lax
from jax.experimental import pallas as pl
from jax.experimental.pallas import tpu as pltpu
from jax.experimental.pallas import tpu_sc as plsc

N_NODES = 10000
D_FEAT = 128
N_EDGES = 320000
MIN_NORM = 1e-15
EPS = 4e-3

NC = 2
NS = 16
NW = NC * NS
K = 80
NB = 125
NBR = NW * NB
RPT = 632
RPT_LAST = N_NODES - RPT * (NS - 1)
DV = D_FEAT // 16
GPB = K // 16

assert NB % 4 == 1 and NB * K * NW == N_EDGES and RPT_LAST % 8 == 0


def _spmm_body(table, edata, out, accum, *scr):
    ebs = scr[0:4]
    rbs = scr[4:8]
    gs = scr[8:12]
    ss = scr[12:16]
    ts = scr[16:20]
    c = lax.axis_index("c")
    s = lax.axis_index("s")
    wid = c * NS + s
    base0 = wid * NB
    r0 = s * RPT

    def stage(b, j):
        r = jnp.minimum(base0 + b, NBR - 1)
        pltpu.async_copy(edata.at[r], ebs[j], ts[j])

    def stwait(j):
        pltpu.make_async_copy(edata.at[base0], ebs[j], ts[j]).wait()

    def gather(j):
        pltpu.async_copy(table.at[ebs[j].at[0]], rbs[j], gs[j])

    def gwait(j):
        pltpu.make_async_copy(table.at[ebs[j].at[0]], rbs[j], gs[j]).wait()

    def scatter(j):
        pltpu.async_copy(rbs[j], accum.at[ebs[j].at[1]], ss[j], add=True)

    def swait(j):
        pltpu.make_async_copy(rbs[j], accum.at[ebs[j].at[1]], ss[j]).wait()

    def scale(j):
        ebb, rows = ebs[j], rbs[j]

        def body(g, inner):
            w16 = lax.bitcast_convert_type(ebb[2, pl.ds(g * 16, 16)],
                                           jnp.float32)
            for l in range(16):
                wl = w16[l]
                for d in range(DV):
                    sl = pl.ds(d * 16, 16)
                    rows[g * 16 + l, sl] = rows[g * 16 + l, sl] * wl
            return inner

        lax.fori_loop(0, GPB, body, 0)

    stage(0, 0)
    stage(1, 1)
    stage(2, 2)
    stwait(0)
    gather(0)
    stwait(1)
    gather(1)

    zvec = jnp.zeros((16,), jnp.float32)
    zb = rbs[3]

    def zero_row(i, carry):
        for d in range(DV):
            zb[i, pl.ds(d * 16, 16)] = zvec
        return carry

    lax.fori_loop(0, K, zero_row, 0)

    @pl.when(s < NS - 1)
    def _():
        for j in range(RPT // K):
            pltpu.sync_copy(zb, accum.at[pl.ds(r0 + j * K, K)])
        pltpu.sync_copy(zb.at[pl.ds(0, RPT % K)],
                        accum.at[pl.ds(r0 + (RPT // K) * K, RPT % K)])

    @pl.when(s == NS - 1)
    def _():
        for j in range(RPT_LAST // K):
            pltpu.sync_copy(zb, accum.at[pl.ds(r0 + j * K, K)])
        pltpu.sync_copy(zb.at[pl.ds(0, RPT_LAST % K)],
                        accum.at[pl.ds(r0 + (RPT_LAST // K) * K,
                                       RPT_LAST % K)])

    plsc.subcore_barrier()

    def step(b, j, first):
        gwait(j)
        scale(j)
        scatter(j)
        j1 = (j + 3) % 4
        if not first:
            swait(j1)
        stage(b + 3, j1)
        j2 = (j + 2) % 4
        stwait(j2)
        gather(j2)

    step(0, 0, first=True)

    def quad(i, carry):
        b = 4 * i + 1
        step(b, 1, False)
        step(b + 1, 2, False)
        step(b + 2, 3, False)
        step(b + 3, 0, False)
        return carry

    lax.fori_loop(0, (NB - 1) // 4, quad, 0)

    gwait(NB % 4)
    gwait((NB + 1) % 4)
    swait((NB - 1) % 4)
    stwait((NB + 2) % 4)
    plsc.subcore_barrier()

    @pl.when(s < NS - 1)
    def _():
        pltpu.sync_copy(accum.at[pl.ds(r0, RPT)], out.at[c, pl.ds(r0, RPT)])

    @pl.when(s == NS - 1)
    def _():
        pltpu.sync_copy(accum.at[pl.ds(r0, RPT_LAST)],
                        out.at[c, pl.ds(r0, RPT_LAST)])


@functools.cache
def _make_spmm():
    return pl.kernel(
        _spmm_body,
        out_type=jax.ShapeDtypeStruct((NC, N_NODES, D_FEAT), jnp.float32),
        mesh=plsc.VectorSubcoreMesh(core_axis_name="c", subcore_axis_name="s",
                                    num_cores=NC, num_subcores=NS),
        scratch_types=(
            [pltpu.VMEM_SHARED((N_NODES, D_FEAT), jnp.float32)]
            + [pltpu.VMEM((3, K), jnp.int32) for _ in range(4)]
            + [pltpu.VMEM((K, D_FEAT), jnp.float32) for _ in range(4)]
            + [pltpu.SemaphoreType.DMA for _ in range(12)]
        ),
    )


def _spmm(table, edata):
    return _make_spmm()(table, edata)


def _logmap0_body(x_ref, o_ref):
    x = x_ref[...]
    norm = jnp.maximum(jnp.sqrt(jnp.sum(x * x, axis=1, keepdims=True)),
                       MIN_NORM)
    z = jnp.clip(norm, -1 + 1e-7, 1 - 1e-7)
    o_ref[...] = (0.5 * jnp.log((1 + z) / (1 - z)) / norm) * x


def _combine_body(p_ref, o_ref):
    o_ref[...] = p_ref[0] + p_ref[1]


def _finish_body(p_ref, o_ref):
    u = p_ref[0] + p_ref[1]
    un = jnp.maximum(jnp.sqrt(jnp.sum(u * u, axis=1, keepdims=True)), MIN_NORM)
    g = jnp.tanh(un) * u / un
    gn = jnp.maximum(jnp.sqrt(jnp.sum(g * g, axis=1, keepdims=True)), MIN_NORM)
    maxnorm = 1.0 - EPS
    o_ref[...] = jnp.where(gn > maxnorm, g / gn * maxnorm, g)


_BR = 1000
_row_spec = pl.BlockSpec((_BR, D_FEAT), lambda i: (i, 0))
_pair_spec = pl.BlockSpec((NC, _BR, D_FEAT), lambda i: (0, i, 0))
_row_shape = jax.ShapeDtypeStruct((N_NODES, D_FEAT), jnp.float32)

_logmap0 = pl.pallas_call(
    _logmap0_body, grid=(N_NODES // _BR,),
    in_specs=[_row_spec], out_specs=_row_spec, out_shape=_row_shape)

_combine = pl.pallas_call(
    _combine_body, grid=(N_NODES // _BR,),
    in_specs=[_pair_spec], out_specs=_row_spec, out_shape=_row_shape)

_finish = pl.pallas_call(
    _finish_body, grid=(N_NODES // _BR,),
    in_specs=[_pair_spec], out_specs=_row_spec, out_shape=_row_shape)


def kernel(x, edge_index, edge_weight):
    src = edge_index[0].astype(jnp.int32)
    dst = edge_index[1].astype(jnp.int32)
    wbits = lax.bitcast_convert_type(edge_weight.astype(jnp.float32),
                                     jnp.int32)
    edata = (jnp.stack([src, dst, wbits], axis=0)
             .reshape(3, NBR, K).transpose(1, 0, 2))

    t = _logmap0(x)
    p1 = _spmm(t, edata)
    y1 = _combine(p1)
    p2 = _spmm(y1, edata)
    return _finish(p2)
